# Initial kernel scaffold; baseline (speedup 1.0000x reference)
import jax
import jax.numpy as jnp
from jax.experimental import pallas as pl


def kernel(node_values, context_indices, context_row_splits):
    raise NotImplementedError("write your pallas kernel here")


# speedup vs baseline: 1.2770x; 1.2770x over previous
import functools

import jax
import jax.numpy as jnp
from jax import lax
from jax.experimental import pallas as pl
from jax.experimental.pallas import tpu as pltpu
from jax.experimental.pallas import tpu_sc as plsc

TOTAL_CTX = 32768
NODE_DIM = 256
NUM_CORES = 2
NUM_SUBCORES = 16
NUM_WORKERS = NUM_CORES * NUM_SUBCORES

ROWS_PER_WORKER = TOTAL_CTX // NUM_WORKERS
CHUNK = 128
NUM_CHUNKS = ROWS_PER_WORKER // CHUNK


def _make_gather():
    mesh = plsc.VectorSubcoreMesh(
        core_axis_name="c", subcore_axis_name="s",
        num_cores=NUM_CORES, num_subcores=NUM_SUBCORES,
    )

    @functools.partial(
        pl.kernel,
        mesh=mesh,
        out_type=jax.ShapeDtypeStruct((TOTAL_CTX, NODE_DIM), jnp.float32),
        scratch_types=[
            pltpu.VMEM((CHUNK,), jnp.int32),
            pltpu.VMEM((CHUNK, NODE_DIM), jnp.float32),
            pltpu.SemaphoreType.DMA,
        ],
    )
    def gather_kernel(table_hbm, idx_hbm, out_hbm, idx_v, rows_v, sem):
        wid = lax.axis_index("s") * NUM_CORES + lax.axis_index("c")
        base = wid * ROWS_PER_WORKER
        for c in range(NUM_CHUNKS):
            off = base + c * CHUNK
            pltpu.sync_copy(idx_hbm.at[pl.ds(off, CHUNK)], idx_v)
            pltpu.async_copy(table_hbm.at[idx_v], rows_v, sem).wait()
            pltpu.sync_copy(rows_v, out_hbm.at[pl.ds(off, CHUNK)])

    return gather_kernel


_gather = _make_gather()


@jax.jit
def kernel(node_values, context_indices, context_row_splits):
    del context_row_splits
    return _gather(node_values, context_indices.astype(jnp.int32))


# speedup vs baseline: 1.4825x; 1.1609x over previous
import functools

import jax
import jax.numpy as jnp
from jax import lax
from jax.experimental import pallas as pl
from jax.experimental.pallas import tpu as pltpu
from jax.experimental.pallas import tpu_sc as plsc

TOTAL_CTX = 32768
NODE_DIM = 256
NUM_CORES = 2
NUM_SUBCORES = 16
NUM_WORKERS = NUM_CORES * NUM_SUBCORES

ROWS_PER_WORKER = TOTAL_CTX // NUM_WORKERS
CHUNK = 128
NUM_CHUNKS = ROWS_PER_WORKER // CHUNK


def _make_gather():
    mesh = plsc.VectorSubcoreMesh(
        core_axis_name="c", subcore_axis_name="s",
        num_cores=NUM_CORES, num_subcores=NUM_SUBCORES,
    )

    @functools.partial(
        pl.kernel,
        mesh=mesh,
        out_type=jax.ShapeDtypeStruct((TOTAL_CTX, NODE_DIM), jnp.float32),
        scratch_types=[
            pltpu.VMEM((CHUNK,), jnp.int32),
            pltpu.VMEM((CHUNK,), jnp.int32),
            pltpu.VMEM((CHUNK, NODE_DIM), jnp.float32),
            pltpu.VMEM((CHUNK, NODE_DIM), jnp.float32),
            pltpu.SemaphoreType.DMA,
            pltpu.SemaphoreType.DMA,
            pltpu.SemaphoreType.DMA,
            pltpu.SemaphoreType.DMA,
        ],
    )
    def gather_kernel(table_hbm, idx_hbm, out_hbm,
                      idx0, idx1, rows0, rows1, g0, g1, w0, w1):
        wid = lax.axis_index("s") * NUM_CORES + lax.axis_index("c")
        base = wid * ROWS_PER_WORKER
        idx_v = [idx0, idx1]
        rows_v = [rows0, rows1]
        gsem = [g0, g1]
        wsem = [w0, w1]
        gather_d = [None, None]
        write_d = [None, None]

        pltpu.sync_copy(idx_hbm.at[pl.ds(base, CHUNK)], idx_v[0])
        gather_d[0] = pltpu.async_copy(table_hbm.at[idx_v[0]], rows_v[0], gsem[0])
        for c in range(NUM_CHUNKS):
            cur = c % 2
            nxt = (c + 1) % 2
            if c + 1 < NUM_CHUNKS:
                if write_d[nxt] is not None:
                    write_d[nxt].wait()
                off = base + (c + 1) * CHUNK
                pltpu.sync_copy(idx_hbm.at[pl.ds(off, CHUNK)], idx_v[nxt])
                gather_d[nxt] = pltpu.async_copy(
                    table_hbm.at[idx_v[nxt]], rows_v[nxt], gsem[nxt])
            gather_d[cur].wait()
            off = base + c * CHUNK
            write_d[cur] = pltpu.async_copy(
                rows_v[cur], out_hbm.at[pl.ds(off, CHUNK)], wsem[cur])
        write_d[(NUM_CHUNKS - 2) % 2].wait()
        write_d[(NUM_CHUNKS - 1) % 2].wait()

    return gather_kernel


_gather = _make_gather()


@jax.jit
def kernel(node_values, context_indices, context_row_splits):
    del context_row_splits
    return _gather(node_values, context_indices.astype(jnp.int32))
